# Initial kernel scaffold; baseline (speedup 1.0000x reference)
import jax
import jax.numpy as jnp
from jax.experimental import pallas as pl


def kernel(x, edge_index, W, b):
    raise NotImplementedError("write your pallas kernel here")


# speedup vs baseline: 29.4117x; 29.4117x over previous
import functools

import jax
import jax.numpy as jnp
from jax import lax
from jax.experimental import pallas as pl
from jax.experimental.pallas import tpu as pltpu
from jax.experimental.pallas import tpu_sc as plsc

NC = 2
NS = 16
NW = NC * NS
CHUNK = 80
BLK = 1000


def _mesh():
    return plsc.VectorSubcoreMesh(core_axis_name="c", subcore_axis_name="s")


def _deg_kernel(n, cpw):
    @functools.partial(
        pl.kernel,
        out_type=jax.ShapeDtypeStruct((NC, n), jnp.float32),
        mesh=_mesh(),
        scratch_types=[
            pltpu.VMEM((2, CHUNK), jnp.int32),
            pltpu.VMEM((2, CHUNK), jnp.int32),
            pltpu.VMEM((CHUNK,), jnp.float32),
            pltpu.VMEM_SHARED((n,), jnp.float32),
            pltpu.SemaphoreType.DMA,
            pltpu.SemaphoreType.DMA,
        ],
    )
    def deg(epk, zeros1, out, ib0, ib1, ones_v, acc, is0, is1):
        c = lax.axis_index("c")
        s = lax.axis_index("s")
        wid = s * NC + c
        base = wid * cpw
        for i in range(CHUNK // 16):
            ones_v[pl.ds(i * 16, 16)] = jnp.full((16,), 1.0, jnp.float32)

        @pl.when(s == 0)
        def _():
            pltpu.sync_copy(zeros1, acc)

        plsc.subcore_barrier()
        ibs = (ib0, ib1)
        sems = (is0, is1)
        pltpu.async_copy(epk.at[base], ib0, is0)
        pltpu.async_copy(epk.at[base + 1], ib1, is1)

        def pair(k2, carry):
            for j in range(2):
                k = k2 * 2 + j
                ib, sem = ibs[j], sems[j]
                pltpu.make_async_copy(epk.at[base + k], ib, sem).wait()
                pltpu.sync_copy(ones_v, acc.at[ib.at[1]], add=True)

                @pl.when(k + 2 < cpw)
                def _():
                    pltpu.async_copy(epk.at[base + k + 2], ib, sem)

            return carry

        lax.fori_loop(0, cpw // 2, pair, 0)
        if cpw % 2:
            k = cpw - 1
            pltpu.make_async_copy(epk.at[base + k], ib0, is0).wait()
            pltpu.sync_copy(ones_v, acc.at[ib0.at[1]], add=True)
        plsc.subcore_barrier()

        @pl.when(s == 0)
        def _():
            pltpu.sync_copy(acc, out.at[c])

    return deg


def _edge_kernel(n, d, cpw):
    @functools.partial(
        pl.kernel,
        out_type=jax.ShapeDtypeStruct((NC, n, d), jnp.float32),
        mesh=_mesh(),
        scratch_types=[
            pltpu.VMEM((2, CHUNK), jnp.int32),
            pltpu.VMEM((2, CHUNK), jnp.int32),
            pltpu.VMEM((CHUNK, d), jnp.float32),
            pltpu.VMEM((CHUNK, d), jnp.float32),
            pltpu.VMEM_SHARED((n, d), jnp.float32),
            pltpu.SemaphoreType.DMA,
            pltpu.SemaphoreType.DMA,
            pltpu.SemaphoreType.DMA,
            pltpu.SemaphoreType.DMA,
        ],
    )
    def edge(epk, g_hbm, zeros2, out, ib0, ib1, rows0, rows1, acc,
             is0, is1, gs0, gs1):
        c = lax.axis_index("c")
        s = lax.axis_index("s")
        wid = s * NC + c
        base = wid * cpw
        rpt = n // 10

        @pl.when(s < 10)
        def _():
            pltpu.sync_copy(zeros2.at[pl.ds(s * rpt, rpt)],
                            acc.at[pl.ds(s * rpt, rpt)])

        plsc.subcore_barrier()
        ibs = (ib0, ib1)
        isems = (is0, is1)
        rows = (rows0, rows1)
        gsems = (gs0, gs1)
        pltpu.async_copy(epk.at[base], ib0, is0)
        pltpu.async_copy(epk.at[base + 1], ib1, is1)
        pltpu.make_async_copy(epk.at[base], ib0, is0).wait()
        pltpu.async_copy(g_hbm.at[ib0.at[0]], rows0, gs0)

        def pair(k2, carry):
            for j in range(2):
                k = k2 * 2 + j
                ib, isem, rw, gsem = ibs[j], isems[j], rows[j], gsems[j]
                ibn, isemn, rwn, gsemn = (ibs[1 - j], isems[1 - j],
                                          rows[1 - j], gsems[1 - j])
                pltpu.make_async_copy(g_hbm.at[ib.at[0]], rw, gsem).wait()

                @pl.when(k + 1 < cpw)
                def _():
                    pltpu.make_async_copy(epk.at[base + k + 1], ibn, isemn).wait()
                    pltpu.async_copy(g_hbm.at[ibn.at[0]], rwn, gsemn)

                pltpu.sync_copy(rw, acc.at[ib.at[1]], add=True)

                @pl.when(k + 2 < cpw)
                def _():
                    pltpu.async_copy(epk.at[base + k + 2], ib, isem)

            return carry

        lax.fori_loop(0, cpw // 2, pair, 0)
        if cpw % 2:
            pltpu.make_async_copy(g_hbm.at[ib0.at[0]], rows0, gs0).wait()
            pltpu.sync_copy(rows0, acc.at[ib0.at[1]], add=True)
        plsc.subcore_barrier()

        @pl.when(s < 10)
        def _():
            pltpu.sync_copy(acc.at[pl.ds(s * rpt, rpt)],
                            out.at[c, pl.ds(s * rpt, rpt)])

    return edge


def _scale_body(xr, wr, degr, gr):
    dinv = lax.rsqrt(degr[0] + degr[1] + 1.0)
    h = jnp.dot(xr[...], wr[...], preferred_element_type=jnp.float32)
    gr[...] = h * dinv


def _scale_kernel(n, d):
    return pl.pallas_call(
        _scale_body,
        grid=(n // BLK,),
        in_specs=[
            pl.BlockSpec((BLK, d), lambda i: (i, 0)),
            pl.BlockSpec((d, d), lambda i: (0, 0)),
            pl.BlockSpec((NC, BLK, 1), lambda i: (0, i, 0)),
        ],
        out_specs=pl.BlockSpec((BLK, d), lambda i: (i, 0)),
        out_shape=jax.ShapeDtypeStruct((n, d), jnp.float32),
    )


def _combine_body(pr, gr, degr, br, outr):
    dinv = lax.rsqrt(degr[0] + degr[1] + 1.0)
    acc = pr[0] + pr[1] + gr[...]
    outr[...] = jnp.maximum(dinv * acc + br[...], 0.0)


def _combine_kernel(n, d):
    return pl.pallas_call(
        _combine_body,
        grid=(n // BLK,),
        in_specs=[
            pl.BlockSpec((NC, BLK, d), lambda i: (0, i, 0)),
            pl.BlockSpec((BLK, d), lambda i: (i, 0)),
            pl.BlockSpec((NC, BLK, 1), lambda i: (0, i, 0)),
            pl.BlockSpec((1, d), lambda i: (0, 0)),
        ],
        out_specs=pl.BlockSpec((BLK, d), lambda i: (i, 0)),
        out_shape=jax.ShapeDtypeStruct((n, d), jnp.float32),
    )


def kernel(x, edge_index, W, b):
    n, d = x.shape
    e = edge_index.shape[1]
    assert e % (NW * CHUNK) == 0 and n % NS == 0 and n % BLK == 0
    cpw = e // (NW * CHUNK)
    chunks = e // CHUNK
    epk = edge_index.reshape(2, chunks, CHUNK).transpose(1, 0, 2)
    zeros1 = jnp.zeros((n,), jnp.float32)
    zeros2 = jnp.zeros((n, d), jnp.float32)

    degp = _deg_kernel(n, cpw)(epk, zeros1)
    degp3 = degp.reshape(NC, n, 1)
    g = _scale_kernel(n, d)(x, W, degp3)
    p = _edge_kernel(n, d, cpw)(epk, g, zeros2)
    return _combine_kernel(n, d)(p, g, degp3, b.reshape(1, d))


# speedup vs baseline: 37.9033x; 1.2887x over previous
import functools

import jax
import jax.numpy as jnp
from jax import lax
from jax.experimental import pallas as pl
from jax.experimental.pallas import tpu as pltpu
from jax.experimental.pallas import tpu_sc as plsc

NC = 2
NS = 16
NW = NC * NS
CHUNK = 128
BLK = 1000


def _mesh():
    return plsc.VectorSubcoreMesh(core_axis_name="c", subcore_axis_name="s")


def _worker_chunks(wid, base_cnt, nx):
    base = base_cnt * wid + 2 * jnp.maximum(wid - nx, 0)
    count = base_cnt + 2 * (wid >= nx).astype(jnp.int32)
    return base, count


def _deg_kernel(n, base_cnt, nx):
    @functools.partial(
        pl.kernel,
        out_type=jax.ShapeDtypeStruct((NC, n), jnp.float32),
        mesh=_mesh(),
        scratch_types=[
            pltpu.VMEM((2, CHUNK), jnp.int32),
            pltpu.VMEM((2, CHUNK), jnp.int32),
            pltpu.VMEM((CHUNK,), jnp.float32),
            pltpu.VMEM_SHARED((n,), jnp.float32),
            pltpu.SemaphoreType.DMA,
            pltpu.SemaphoreType.DMA,
        ],
    )
    def deg(epk, zeros1, out, ib0, ib1, ones_v, acc, is0, is1):
        c = lax.axis_index("c")
        s = lax.axis_index("s")
        wid = s * NC + c
        base, count = _worker_chunks(wid, base_cnt, nx)
        for i in range(CHUNK // 16):
            ones_v[pl.ds(i * 16, 16)] = jnp.full((16,), 1.0, jnp.float32)

        @pl.when(s == 0)
        def _():
            pltpu.sync_copy(zeros1, acc)

        plsc.subcore_barrier()
        ibs = (ib0, ib1)
        sems = (is0, is1)
        pltpu.async_copy(epk.at[base], ib0, is0)
        pltpu.async_copy(epk.at[base + 1], ib1, is1)

        def pair(k2, carry):
            for j in range(2):
                k = k2 * 2 + j
                ib, sem = ibs[j], sems[j]
                pltpu.make_async_copy(epk.at[base + k], ib, sem).wait()
                pltpu.sync_copy(ones_v, acc.at[ib.at[1]], add=True)

                @pl.when(k + 2 < count)
                def _():
                    pltpu.async_copy(epk.at[base + k + 2], ib, sem)

            return carry

        lax.fori_loop(0, count // 2, pair, 0)
        plsc.subcore_barrier()

        @pl.when(s == 0)
        def _():
            pltpu.sync_copy(acc, out.at[c])

    return deg


def _edge_kernel(n, d, base_cnt, nx):
    @functools.partial(
        pl.kernel,
        out_type=jax.ShapeDtypeStruct((NC, n, d), jnp.float32),
        mesh=_mesh(),
        scratch_types=[
            pltpu.VMEM((2, CHUNK), jnp.int32),
            pltpu.VMEM((2, CHUNK), jnp.int32),
            pltpu.VMEM((CHUNK, d), jnp.float32),
            pltpu.VMEM((CHUNK, d), jnp.float32),
            pltpu.VMEM_SHARED((n, d), jnp.float32),
            pltpu.SemaphoreType.DMA,
            pltpu.SemaphoreType.DMA,
            pltpu.SemaphoreType.DMA,
            pltpu.SemaphoreType.DMA,
        ],
    )
    def edge(epk, g_hbm, zeros2, out, ib0, ib1, rows0, rows1, acc,
             is0, is1, gs0, gs1):
        c = lax.axis_index("c")
        s = lax.axis_index("s")
        wid = s * NC + c
        base, count = _worker_chunks(wid, base_cnt, nx)
        rpt = n // 10

        @pl.when(s < 10)
        def _():
            pltpu.sync_copy(zeros2.at[pl.ds(s * rpt, rpt)],
                            acc.at[pl.ds(s * rpt, rpt)])

        plsc.subcore_barrier()
        ibs = (ib0, ib1)
        isems = (is0, is1)
        rows = (rows0, rows1)
        gsems = (gs0, gs1)
        pltpu.async_copy(epk.at[base], ib0, is0)
        pltpu.async_copy(epk.at[base + 1], ib1, is1)
        pltpu.make_async_copy(epk.at[base], ib0, is0).wait()
        pltpu.async_copy(g_hbm.at[ib0.at[0]], rows0, gs0)

        def pair(k2, carry):
            for j in range(2):
                k = k2 * 2 + j
                ib, isem, rw, gsem = ibs[j], isems[j], rows[j], gsems[j]
                ibn, isemn, rwn, gsemn = (ibs[1 - j], isems[1 - j],
                                          rows[1 - j], gsems[1 - j])
                pltpu.make_async_copy(g_hbm.at[ib.at[0]], rw, gsem).wait()

                @pl.when(k + 1 < count)
                def _():
                    pltpu.make_async_copy(epk.at[base + k + 1], ibn, isemn).wait()
                    pltpu.async_copy(g_hbm.at[ibn.at[0]], rwn, gsemn)

                pltpu.sync_copy(rw, acc.at[ib.at[1]], add=True)

                @pl.when(k + 2 < count)
                def _():
                    pltpu.async_copy(epk.at[base + k + 2], ib, isem)

            return carry

        lax.fori_loop(0, count // 2, pair, 0)
        plsc.subcore_barrier()

        @pl.when(s < 10)
        def _():
            pltpu.sync_copy(acc.at[pl.ds(s * rpt, rpt)],
                            out.at[c, pl.ds(s * rpt, rpt)])

    return edge


def _scale_body(xr, wr, degr, gr):
    dinv = lax.rsqrt(degr[0] + degr[1] + 1.0)
    h = jnp.dot(xr[...], wr[...], preferred_element_type=jnp.float32)
    gr[...] = h * dinv


def _scale_kernel(n, d):
    return pl.pallas_call(
        _scale_body,
        grid=(n // BLK,),
        in_specs=[
            pl.BlockSpec((BLK, d), lambda i: (i, 0)),
            pl.BlockSpec((d, d), lambda i: (0, 0)),
            pl.BlockSpec((NC, BLK, 1), lambda i: (0, i, 0)),
        ],
        out_specs=pl.BlockSpec((BLK, d), lambda i: (i, 0)),
        out_shape=jax.ShapeDtypeStruct((n, d), jnp.float32),
    )


def _combine_body(pr, gr, degr, br, outr):
    dinv = lax.rsqrt(degr[0] + degr[1] + 1.0)
    acc = pr[0] + pr[1] + gr[...]
    outr[...] = jnp.maximum(dinv * acc + br[...], 0.0)


def _combine_kernel(n, d):
    return pl.pallas_call(
        _combine_body,
        grid=(n // BLK,),
        in_specs=[
            pl.BlockSpec((NC, BLK, d), lambda i: (0, i, 0)),
            pl.BlockSpec((BLK, d), lambda i: (i, 0)),
            pl.BlockSpec((NC, BLK, 1), lambda i: (0, i, 0)),
            pl.BlockSpec((1, d), lambda i: (0, 0)),
        ],
        out_specs=pl.BlockSpec((BLK, d), lambda i: (i, 0)),
        out_shape=jax.ShapeDtypeStruct((n, d), jnp.float32),
    )


def kernel(x, edge_index, W, b):
    n, d = x.shape
    e = edge_index.shape[1]
    assert e % CHUNK == 0 and n % BLK == 0
    chunks = e // CHUNK
    base_cnt = (chunks // NW) & ~1
    extra = chunks - base_cnt * NW
    assert extra % 2 == 0 and extra // 2 <= NW
    nx = NW - extra // 2
    epk = edge_index.reshape(2, chunks, CHUNK).transpose(1, 0, 2)
    zeros1 = jnp.zeros((n,), jnp.float32)
    zeros2 = jnp.zeros((n, d), jnp.float32)

    degp = _deg_kernel(n, base_cnt, nx)(epk, zeros1)
    degp3 = degp.reshape(NC, n, 1)
    g = _scale_kernel(n, d)(x, W, degp3)
    p = _edge_kernel(n, d, base_cnt, nx)(epk, g, zeros2)
    return _combine_kernel(n, d)(p, g, degp3, b.reshape(1, d))
